# Initial kernel scaffold; baseline (speedup 1.0000x reference)
import jax
import jax.numpy as jnp
from jax.experimental import pallas as pl


def kernel(x, codebooks):
    raise NotImplementedError("write your pallas kernel here")


# speedup vs baseline: 1.3178x; 1.3178x over previous
import functools

import jax
import jax.numpy as jnp
from jax.experimental import pallas as pl

NUM_Q = 4
CB_DIM = 64
CB_SIZE = 1024
BETA = 0.25
B, H, W = 8, 32, 32
N = H * W


def _vq_kernel(x_ref, cb_ref, quant_ref, idx_ref, hist_ref, loss_ref,
               perp_ref):
    b = pl.program_id(1)
    xb = x_ref[0, 0]
    cb = cb_ref[0]

    cbnorm = jnp.sum(cb * cb, axis=1, keepdims=True)
    znorm = jnp.sum(xb * xb, axis=0, keepdims=True)
    dots = jax.lax.dot(cb, xb)
    scores = (znorm - 2.0 * dots) + cbnorm

    m = jnp.min(scores, axis=0, keepdims=True)
    iota_k = jax.lax.broadcasted_iota(jnp.int32, (CB_SIZE, 1), 0)
    idx = jnp.min(jnp.where(scores == m, iota_k, CB_SIZE), axis=0,
                  keepdims=True)
    onehot = (iota_k == idx).astype(jnp.float32)

    quant_ref[0, 0] = jax.lax.dot(
        cb.T, onehot, precision=jax.lax.Precision.HIGHEST)
    idx_ref[0, 0] = idx

    hist_c = jnp.sum(onehot, axis=1, keepdims=True).T
    loss_c = jnp.sum(m)

    @pl.when(b == 0)
    def _init():
        hist_ref[0] = hist_c
        loss_ref[0] = jnp.full((1, 128), loss_c, jnp.float32)

    @pl.when(b > 0)
    def _acc():
        hist_ref[0] = hist_ref[0] + hist_c
        loss_ref[0] = loss_ref[0] + loss_c

    @pl.when(b == B - 1)
    def _finalize():
        hist = hist_ref[0]
        probs = hist * (1.0 / (B * N))
        ent = jnp.sum(probs * jnp.log(probs + 1e-10))
        perp_ref[0] = jnp.full((1, 128), jnp.exp(-ent), jnp.float32)
        loss_ref[0] = loss_ref[0] * ((1.0 + BETA) / (B * N * CB_DIM))


@jax.jit
def kernel(x, codebooks):
    xr = x.reshape(B, NUM_Q, CB_DIM, N)
    quant, idx, hist, loss, perp = pl.pallas_call(
        _vq_kernel,
        grid=(NUM_Q, B),
        in_specs=[
            pl.BlockSpec((1, 1, CB_DIM, N), lambda i, b: (b, i, 0, 0)),
            pl.BlockSpec((1, CB_SIZE, CB_DIM), lambda i, b: (i, 0, 0)),
        ],
        out_specs=[
            pl.BlockSpec((1, 1, CB_DIM, N), lambda i, b: (b, i, 0, 0)),
            pl.BlockSpec((1, 1, 1, N), lambda i, b: (b, i, 0, 0)),
            pl.BlockSpec((1, 1, CB_SIZE), lambda i, b: (i, 0, 0)),
            pl.BlockSpec((1, 1, 128), lambda i, b: (i, 0, 0)),
            pl.BlockSpec((1, 1, 128), lambda i, b: (i, 0, 0)),
        ],
        out_shape=[
            jax.ShapeDtypeStruct((B, NUM_Q, CB_DIM, N), jnp.float32),
            jax.ShapeDtypeStruct((B, NUM_Q, 1, N), jnp.int32),
            jax.ShapeDtypeStruct((NUM_Q, 1, CB_SIZE), jnp.float32),
            jax.ShapeDtypeStruct((NUM_Q, 1, 128), jnp.float32),
            jax.ShapeDtypeStruct((NUM_Q, 1, 128), jnp.float32),
        ],
    )(xr, codebooks)
    quantized_cat = quant.reshape(B, NUM_Q * CB_DIM, H, W)
    indices_cat = idx.reshape(B, NUM_Q, H, W)
    loss_cat = loss[:, 0, 0]
    perplexity_cat = perp[:, 0, 0]
    return (quantized_cat, indices_cat, loss_cat, perplexity_cat)


# speedup vs baseline: 2.0604x; 1.5635x over previous
import functools

import jax
import jax.numpy as jnp
from jax.experimental import pallas as pl

NUM_Q = 4
CB_DIM = 64
CB_SIZE = 1024
BETA = 0.25
B, H, W = 8, 32, 32
N = H * W


def _vq_kernel(x_ref, cb_ref, quant_ref, idx_ref, hist_ref, loss_ref,
               perp_ref):
    b = pl.program_id(1)
    xb = x_ref[0, 0]
    cb = cb_ref[0]

    cbnorm = jnp.sum(cb * cb, axis=1, keepdims=True)
    znorm = jnp.sum(xb * xb, axis=0, keepdims=True)
    dots = jax.lax.dot(cb, xb)
    scores = (znorm - 2.0 * dots) + cbnorm

    m = jnp.min(scores, axis=0, keepdims=True)
    iota_k = jax.lax.broadcasted_iota(jnp.int32, (CB_SIZE, 1), 0)
    idx = jnp.argmin(scores, axis=0)[None, :]
    onehot = (iota_k == idx).astype(jnp.float32)

    quant_ref[0, 0] = jax.lax.dot(
        cb.T.astype(jnp.bfloat16), onehot.astype(jnp.bfloat16),
        preferred_element_type=jnp.float32)
    idx_ref[0, 0] = idx

    hist_c = jnp.sum(onehot, axis=1, keepdims=True).T
    loss_c = jnp.sum(m)

    @pl.when(b == 0)
    def _init():
        hist_ref[0] = hist_c
        loss_ref[0] = jnp.full((1, 128), loss_c, jnp.float32)

    @pl.when(b > 0)
    def _acc():
        hist_ref[0] = hist_ref[0] + hist_c
        loss_ref[0] = loss_ref[0] + loss_c

    @pl.when(b == B - 1)
    def _finalize():
        hist = hist_ref[0]
        probs = hist * (1.0 / (B * N))
        ent = jnp.sum(probs * jnp.log(probs + 1e-10))
        perp_ref[0] = jnp.full((1, 128), jnp.exp(-ent), jnp.float32)
        loss_ref[0] = loss_ref[0] * ((1.0 + BETA) / (B * N * CB_DIM))


@jax.jit
def kernel(x, codebooks):
    xr = x.reshape(B, NUM_Q, CB_DIM, N)
    quant, idx, hist, loss, perp = pl.pallas_call(
        _vq_kernel,
        grid=(NUM_Q, B),
        in_specs=[
            pl.BlockSpec((1, 1, CB_DIM, N), lambda i, b: (b, i, 0, 0)),
            pl.BlockSpec((1, CB_SIZE, CB_DIM), lambda i, b: (i, 0, 0)),
        ],
        out_specs=[
            pl.BlockSpec((1, 1, CB_DIM, N), lambda i, b: (b, i, 0, 0)),
            pl.BlockSpec((1, 1, 1, N), lambda i, b: (b, i, 0, 0)),
            pl.BlockSpec((1, 1, CB_SIZE), lambda i, b: (i, 0, 0)),
            pl.BlockSpec((1, 1, 128), lambda i, b: (i, 0, 0)),
            pl.BlockSpec((1, 1, 128), lambda i, b: (i, 0, 0)),
        ],
        out_shape=[
            jax.ShapeDtypeStruct((B, NUM_Q, CB_DIM, N), jnp.float32),
            jax.ShapeDtypeStruct((B, NUM_Q, 1, N), jnp.int32),
            jax.ShapeDtypeStruct((NUM_Q, 1, CB_SIZE), jnp.float32),
            jax.ShapeDtypeStruct((NUM_Q, 1, 128), jnp.float32),
            jax.ShapeDtypeStruct((NUM_Q, 1, 128), jnp.float32),
        ],
    )(xr, codebooks)
    quantized_cat = quant.reshape(B, NUM_Q * CB_DIM, H, W)
    indices_cat = idx.reshape(B, NUM_Q, H, W)
    loss_cat = loss[:, 0, 0]
    perplexity_cat = perp[:, 0, 0]
    return (quantized_cat, indices_cat, loss_cat, perplexity_cat)
